# Initial kernel scaffold; baseline (speedup 1.0000x reference)
import jax
import jax.numpy as jnp
from jax.experimental import pallas as pl


def kernel(ft, fs):
    raise NotImplementedError("write your pallas kernel here")


# speedup vs baseline: 6.9407x; 6.9407x over previous
import jax
import jax.numpy as jnp
from jax.experimental import pallas as pl
from jax.experimental.pallas import tpu as pltpu

_N = 2048
_D = 128
_OT_ITER = 20


def _gnncom_kernel(ft_ref, fs_ref, loss_ref, p_ref, m_ref):
    ft = ft_ref[...]
    fs = fs_ref[...]

    ftn = ft / jnp.maximum(
        jnp.sqrt(jnp.sum(ft * ft, axis=1, keepdims=True)), 1e-12)
    fsn = fs / jnp.maximum(
        jnp.sqrt(jnp.sum(fs * fs, axis=1, keepdims=True)), 1e-12)

    m = jax.lax.dot_general(
        ftn, fsn,
        dimension_numbers=(((1,), (1,)), ((), ())),
        preferred_element_type=jnp.float32)

    mn_lo = jnp.min(m)
    mn_hi = jnp.max(m)
    m = (m - mn_lo) / (mn_hi - mn_lo)
    m_ref[...] = m

    rowmax = jnp.max(m, axis=1, keepdims=True)
    k = jnp.exp(m - rowmax)

    r = 1.0 / _N
    c = 1.0 / _N
    u0 = jnp.zeros((_N, 1), dtype=jnp.float32)
    v0 = jnp.ones((1, _N), dtype=jnp.float32)

    def body(_, uv):
        _, v = uv
        kv = jax.lax.dot_general(
            k, v, dimension_numbers=(((1,), (1,)), ((), ())),
            preferred_element_type=jnp.float32)
        u = r / kv
        ktu = jax.lax.dot_general(
            u, k, dimension_numbers=(((0,), (0,)), ((), ())),
            preferred_element_type=jnp.float32)
        v = c / ktu
        return (u, v)

    u, v = jax.lax.fori_loop(0, _OT_ITER, body, (u0, v0))

    kv = jax.lax.dot_general(
        k, v, dimension_numbers=(((1,), (1,)), ((), ())),
        preferred_element_type=jnp.float32)
    u = 1.0 / kv
    ktu = jax.lax.dot_general(
        u, k, dimension_numbers=(((0,), (0,)), ((), ())),
        preferred_element_type=jnp.float32)
    v = 1.0 / ktu

    p = u * k * v
    p_ref[...] = p

    row_i = jax.lax.broadcasted_iota(jnp.int32, (_N, _N), 0)
    col_i = jax.lax.broadcasted_iota(jnp.int32, (_N, _N), 1)
    diag = jnp.sum(jnp.where(row_i == col_i, p, 0.0), keepdims=True)
    psq = jnp.sum(p * p, keepdims=True)
    loss_ref[...] = jnp.sqrt(psq - 2.0 * diag + jnp.float32(_N))


def kernel(ft, fs):
    loss2d, p, m = pl.pallas_call(
        _gnncom_kernel,
        out_shape=[
            jax.ShapeDtypeStruct((1, 1), jnp.float32),
            jax.ShapeDtypeStruct((_N, _N), jnp.float32),
            jax.ShapeDtypeStruct((_N, _N), jnp.float32),
        ],
        compiler_params=pltpu.CompilerParams(
            vmem_limit_bytes=120 * 1024 * 1024),
    )(ft, fs)
    return (loss2d[0, 0], p, m)
